# baseline (speedup 1.0000x reference)
import functools

import jax
import jax.numpy as jnp
from jax import lax
from jax.experimental import pallas as pl
from jax.experimental.pallas import tpu as pltpu
from jax.experimental.pallas import tpu_sc as plsc

B, T, V, D = 4096, 200, 1000, 64
L = 16
NW = 32
BW = B // NW
NO = 2
TH = T // 2
VA = V + T


def _sc_embed(xt, aug_table, pos_table):
    mesh = plsc.VectorSubcoreMesh(core_axis_name="c", subcore_axis_name="s")

    @functools.partial(
        pl.kernel,
        mesh=mesh,
        compiler_params=pltpu.CompilerParams(
            use_tc_tiling_on_sc=False, needs_layout_passes=False),
        out_type=jax.ShapeDtypeStruct((T, D // 8, NW, 8, BW), jnp.float32),
        scratch_types=(
            [pltpu.VMEM((TH, BW), jnp.int32)]
            + [pltpu.VMEM((D // 8, 8, BW + 1), jnp.float32) for _ in range(NO)]
            + [pltpu.VMEM((T, D), jnp.float32)]
            + [pltpu.VMEM((VA, D + 1), jnp.float32)]
            + [pltpu.SemaphoreType.DMA for _ in range(NO)]
        ),
    )
    def k(x_hbm, aug_hbm, pos_hbm, out_hbm, idx_h, o0, o1, pos_v, aug_v,
          s0, s1):
        outs = (o0, o1)
        osem = (s0, s1)
        wid = lax.axis_index("s") * 2 + lax.axis_index("c")
        b0 = wid * BW

        pltpu.sync_copy(aug_hbm, aug_v.at[:, pl.ds(0, D)])
        pltpu.sync_copy(pos_hbm, pos_v)
        pltpu.sync_copy(x_hbm.at[pl.ds(0, TH), pl.ds(b0, BW)], idx_h)

        iota = lax.iota(jnp.int32, L)
        dr_vec = jnp.where(iota >= 8, iota - 8, iota)
        dt_half = jnp.where(iota >= 8, 1, 0)
        dtv = [dt_half + 2 * g for g in range(D // L)]
        dvec = [iota + g * L for g in range(D // L)]

        def o_desc(t, o):
            return pltpu.make_async_copy(
                outs[o].at[:, :, pl.ds(0, BW)], out_hbm.at[t, :, wid],
                osem[o])

        def gather_transpose_add(t, row, dst):
            pv = [pos_v[t, pl.ds(g * L, L)] for g in range(D // L)]
            tv = jnp.full((L,), V, jnp.int32) + t
            rr = jnp.full((L,), row, jnp.int32)

            def per_j(j, jj):
                iv = plsc.load_gather(idx_h, [rr, jj])
                iv = jnp.where(iv == 2, tv, iv)
                vs = [plsc.load_gather(aug_v, [iv, dvec[g]])
                      for g in range(D // L)]
                ws = [vs[g] + pv[g] for g in range(D // L)]
                for g in range(D // L):
                    plsc.store_scatter(dst, [dtv[g], dr_vec, jj], ws[g])
                return jj + 1

            lax.fori_loop(0, BW, per_j, jnp.zeros((L,), jnp.int32), unroll=4)

        def step(t, row, o):
            @pl.when(t >= NO)
            def _():
                o_desc(t - NO, o).wait()
            gather_transpose_add(t, row, outs[o])
            o_desc(t, o).start()

        def half(h):
            def outer(u, carry):
                t0 = h * TH + u * NO
                for i in range(NO):
                    step(t0 + i, u * NO + i, i)
                return carry
            lax.fori_loop(0, TH // NO, outer, 0)

        half(0)
        pltpu.sync_copy(x_hbm.at[pl.ds(TH, TH), pl.ds(b0, BW)], idx_h)
        half(1)

        for o in range(NO):
            o_desc(T - NO + o, o).wait()

    return k(xt, aug_table, pos_table)


def kernel(x, start_token, end_token, tok_table, pos_table):
    xt = jnp.swapaxes(x, 0, 1)
    aug_table = jnp.concatenate(
        [tok_table, jnp.float32(-5.0) - pos_table], axis=0)
    out5 = _sc_embed(xt, aug_table, pos_table)
    return jnp.transpose(out5, (2, 4, 0, 1, 3)).reshape(B, T, D)


# speedup vs baseline: 1.4691x; 1.4691x over previous
import functools

import jax
import jax.numpy as jnp
from jax import lax
from jax.experimental import pallas as pl
from jax.experimental.pallas import tpu as pltpu
from jax.experimental.pallas import tpu_sc as plsc

B, T, V, D = 4096, 200, 1000, 64
L = 16
NW = 32
BW = B // NW
NO = 2
TH = T // 2
VA = V + T


def _sc_embed(xt, aug_table, pos_table):
    mesh = plsc.VectorSubcoreMesh(core_axis_name="c", subcore_axis_name="s")

    @functools.partial(
        pl.kernel,
        mesh=mesh,
        compiler_params=pltpu.CompilerParams(
            use_tc_tiling_on_sc=False, needs_layout_passes=False),
        out_type=jax.ShapeDtypeStruct((T, D // 8, NW, 8, BW), jnp.float32),
        scratch_types=(
            [pltpu.VMEM((TH, BW), jnp.int32)]
            + [pltpu.VMEM((D // 8, 8, BW + 1), jnp.float32) for _ in range(NO)]
            + [pltpu.VMEM((T, D), jnp.float32)]
            + [pltpu.VMEM((VA, D + 1), jnp.float32)]
            + [pltpu.SemaphoreType.DMA for _ in range(NO)]
        ),
    )
    def k(x_hbm, aug_hbm, pos_hbm, out_hbm, idx_h, o0, o1, pos_v, aug_v,
          s0, s1):
        outs = (o0, o1)
        osem = (s0, s1)
        wid = lax.axis_index("s") * 2 + lax.axis_index("c")
        b0 = wid * BW

        pltpu.sync_copy(aug_hbm, aug_v.at[:, pl.ds(0, D)])
        pltpu.sync_copy(pos_hbm, pos_v)
        pltpu.sync_copy(x_hbm.at[pl.ds(0, TH), pl.ds(b0, BW)], idx_h)

        iota = lax.iota(jnp.int32, L)
        dr_vec = jnp.where(iota >= 8, iota - 8, iota)
        dt_half = jnp.where(iota >= 8, 1, 0)
        dtv = [dt_half + 2 * g for g in range(D // L)]
        dvec = [iota + g * L for g in range(D // L)]

        def o_desc(t, o):
            return pltpu.make_async_copy(
                outs[o].at[:, :, pl.ds(0, BW)], out_hbm.at[t, :, wid],
                osem[o])

        def gather_transpose_add(t, row, dst):
            pv = [pos_v[t, pl.ds(g * L, L)] for g in range(D // L)]
            tv = jnp.full((L,), V, jnp.int32) + t

            def per_blk(j0, jj):
                ivv = idx_h[row, pl.ds(j0 * L, L)]
                ivv = jnp.where(ivv == 2, tv, ivv)
                for l in range(L):
                    iv = jnp.full((L,), ivv[l], jnp.int32)
                    vs = [plsc.load_gather(aug_v, [iv, dvec[g]])
                          for g in range(D // L)]
                    ws = [vs[g] + pv[g] for g in range(D // L)]
                    for g in range(D // L):
                        plsc.store_scatter(dst, [dtv[g], dr_vec, jj], ws[g])
                    jj = jj + 1
                return jj

            lax.fori_loop(0, BW // L, per_blk,
                          jnp.zeros((L,), jnp.int32))

        def step(t, row, o):
            @pl.when(t >= NO)
            def _():
                o_desc(t - NO, o).wait()
            gather_transpose_add(t, row, outs[o])
            o_desc(t, o).start()

        def half(h):
            def outer(u, carry):
                t0 = h * TH + u * NO
                for i in range(NO):
                    step(t0 + i, u * NO + i, i)
                return carry
            lax.fori_loop(0, TH // NO, outer, 0)

        half(0)
        pltpu.sync_copy(x_hbm.at[pl.ds(TH, TH), pl.ds(b0, BW)], idx_h)
        half(1)

        for o in range(NO):
            o_desc(T - NO + o, o).wait()

    return k(xt, aug_table, pos_table)


def kernel(x, start_token, end_token, tok_table, pos_table):
    xt = jnp.swapaxes(x, 0, 1)
    aug_table = jnp.concatenate(
        [tok_table, jnp.float32(-5.0) - pos_table], axis=0)
    out5 = _sc_embed(xt, aug_table, pos_table)
    return jnp.transpose(out5, (2, 4, 0, 1, 3)).reshape(B, T, D)


# speedup vs baseline: 1.5161x; 1.0319x over previous
import functools

import jax
import jax.numpy as jnp
from jax import lax
from jax.experimental import pallas as pl
from jax.experimental.pallas import tpu as pltpu
from jax.experimental.pallas import tpu_sc as plsc

B, T, V, D = 4096, 200, 1000, 64
L = 16
NW = 32
BW = B // NW
NO = 2
TH = T // 2
VA = V + T


def _sc_embed(xt, aug_table, pos_table):
    mesh = plsc.VectorSubcoreMesh(core_axis_name="c", subcore_axis_name="s")

    @functools.partial(
        pl.kernel,
        mesh=mesh,
        compiler_params=pltpu.CompilerParams(
            use_tc_tiling_on_sc=False, needs_layout_passes=False),
        out_type=jax.ShapeDtypeStruct((T, D // 8, NW, 8, BW), jnp.float32),
        scratch_types=(
            [pltpu.VMEM((TH, BW), jnp.int32)]
            + [pltpu.VMEM((D // 8, 8, BW + 1), jnp.float32) for _ in range(NO)]
            + [pltpu.VMEM((T, D), jnp.float32)]
            + [pltpu.VMEM((VA, D + 1), jnp.float32)]
            + [pltpu.SemaphoreType.DMA for _ in range(NO)]
        ),
    )
    def k(x_hbm, aug_hbm, pos_hbm, out_hbm, idx_h, o0, o1, pos_v, aug_v,
          s0, s1):
        outs = (o0, o1)
        osem = (s0, s1)
        wid = lax.axis_index("s") * 2 + lax.axis_index("c")
        b0 = wid * BW

        pltpu.sync_copy(aug_hbm, aug_v.at[:, pl.ds(0, D)])
        pltpu.sync_copy(pos_hbm, pos_v)
        pltpu.sync_copy(x_hbm.at[pl.ds(0, TH), pl.ds(b0, BW)], idx_h)

        iota = lax.iota(jnp.int32, L)
        dr_vec = jnp.where(iota >= 8, iota - 8, iota)
        dt_half = jnp.where(iota >= 8, 1, 0)
        dtv = [dt_half + 2 * g for g in range(D // L)]
        dvec = [iota + g * L for g in range(D // L)]

        def o_desc(t, o):
            return pltpu.make_async_copy(
                outs[o].at[:, :, pl.ds(0, BW)], out_hbm.at[t, :, wid],
                osem[o])

        def gather_transpose_add(t, row, dst):
            pv = [pos_v[t, pl.ds(g * L, L)] for g in range(D // L)]
            tv = jnp.full((L,), V, jnp.int32) + t

            zv = jnp.zeros((L,), jnp.int32)

            def per_blk(j0, carry):
                ivv = idx_h[row, pl.ds(j0 * L, L)]
                ivv = jnp.where(ivv == 2, tv, ivv)
                jbase = j0 * L
                for l in range(L):
                    iv = jnp.full((L,), ivv[l], jnp.int32)
                    jv = zv + (jbase + l)
                    vs = [plsc.load_gather(aug_v, [iv, dvec[g]])
                          for g in range(D // L)]
                    ws = [vs[g] + pv[g] for g in range(D // L)]
                    for g in range(D // L):
                        plsc.store_scatter(dst, [dtv[g], dr_vec, jv], ws[g])
                return carry

            lax.fori_loop(0, BW // L, per_blk, 0)

        def step(t, row, o):
            @pl.when(t >= NO)
            def _():
                o_desc(t - NO, o).wait()
            gather_transpose_add(t, row, outs[o])
            o_desc(t, o).start()

        def half(h):
            def outer(u, carry):
                t0 = h * TH + u * NO
                for i in range(NO):
                    step(t0 + i, u * NO + i, i)
                return carry
            lax.fori_loop(0, TH // NO, outer, 0)

        half(0)
        pltpu.sync_copy(x_hbm.at[pl.ds(TH, TH), pl.ds(b0, BW)], idx_h)
        half(1)

        for o in range(NO):
            o_desc(T - NO + o, o).wait()

    return k(xt, aug_table, pos_table)


def kernel(x, start_token, end_token, tok_table, pos_table):
    xt = jnp.swapaxes(x, 0, 1)
    aug_table = jnp.concatenate(
        [tok_table, jnp.float32(-5.0) - pos_table], axis=0)
    out5 = _sc_embed(xt, aug_table, pos_table)
    return jnp.transpose(out5, (2, 4, 0, 1, 3)).reshape(B, T, D)


# speedup vs baseline: 1.5904x; 1.0490x over previous
import functools

import jax
import jax.numpy as jnp
from jax import lax
from jax.experimental import pallas as pl
from jax.experimental.pallas import tpu as pltpu
from jax.experimental.pallas import tpu_sc as plsc

B, T, V, D = 4096, 200, 1000, 64
L = 16
NW = 32
BW = B // NW
JS = BW // 2
TH = T // 2
VA = V + T


def _sc_embed(xt, aug_table, pos_table):
    mesh = plsc.VectorSubcoreMesh(core_axis_name="c", subcore_axis_name="s")

    @functools.partial(
        pl.kernel,
        mesh=mesh,
        compiler_params=pltpu.CompilerParams(
            use_tc_tiling_on_sc=False, needs_layout_passes=False),
        out_type=jax.ShapeDtypeStruct((T, D // 8, NW, 8, BW), jnp.float32),
        scratch_types=(
            [pltpu.VMEM((TH, BW), jnp.int32)]
            + [pltpu.VMEM((JS, D), jnp.float32) for _ in range(2)]
            + [pltpu.VMEM((D // 8, 8, BW + 1), jnp.float32) for _ in range(2)]
            + [pltpu.VMEM((T, D), jnp.float32)]
            + [pltpu.VMEM((VA, D), jnp.float32)]
            + [pltpu.SemaphoreType.DMA for _ in range(4)]
        ),
    )
    def k(x_hbm, aug_hbm, pos_hbm, out_hbm, idx_h, r0, r1, o0, o1,
          pos_v, aug_v, g0, g1, s0, s1):
        rows = (r0, r1)
        outs = (o0, o1)
        gsem = (g0, g1)
        osem = (s0, s1)
        wid = lax.axis_index("s") * 2 + lax.axis_index("c")
        b0 = wid * BW

        pltpu.sync_copy(aug_hbm, aug_v)
        pltpu.sync_copy(pos_hbm, pos_v)

        iota = lax.iota(jnp.int32, L)
        dr_vec = jnp.where(iota >= 8, iota - 8, iota)
        dt_half = jnp.where(iota >= 8, 1, 0)
        dtv = [dt_half + 2 * g for g in range(D // L)]
        dvec = [iota + g * L for g in range(D // L)]
        zv = jnp.zeros((L,), jnp.int32)

        def stage_half(h):
            pltpu.sync_copy(x_hbm.at[pl.ds(h * TH, TH), pl.ds(b0, BW)],
                            idx_h)

            def remap(r, carry):
                for kk in range(BW // L):
                    v = idx_h[r, pl.ds(kk * L, L)]
                    idx_h[r, pl.ds(kk * L, L)] = jnp.where(
                        v == 2, (V + h * TH) + r, v)
                return carry

            lax.fori_loop(0, TH, remap, 0, unroll=2)

        def g_desc(row, p):
            return pltpu.make_async_copy(
                aug_hbm.at[idx_h.at[row, pl.ds(0, JS)]], rows[p], gsem[p])

        def o_desc(t, o):
            return pltpu.make_async_copy(
                outs[o].at[:, :, pl.ds(0, BW)], out_hbm.at[t, :, wid],
                osem[o])

        def streamed_transpose(src, dst, t):
            pv = [pos_v[t, pl.ds(g * L, L)] for g in range(D // L)]

            def per_j(j, carry):
                jv = zv + j
                vs = [src[j, pl.ds(g * L, L)] for g in range(D // L)]
                ws = [vs[g] + pv[g] for g in range(D // L)]
                for g in range(D // L):
                    plsc.store_scatter(dst, [dtv[g], dr_vec, jv], ws[g])
                return carry

            lax.fori_loop(0, JS, per_j, 0, unroll=4)

        def vldidx_transpose(row, dst, t):
            pv = [pos_v[t, pl.ds(g * L, L)] for g in range(D // L)]

            def per_blk(q, carry):
                j0 = (JS // L) + q
                ivv = idx_h[row, pl.ds(j0 * L, L)]
                for l in range(L):
                    iv = jnp.full((L,), ivv[l], jnp.int32)
                    jv = zv + (j0 * L + l)
                    vs = [plsc.load_gather(aug_v, [iv, dvec[g]])
                          for g in range(D // L)]
                    ws = [vs[g] + pv[g] for g in range(D // L)]
                    for g in range(D // L):
                        plsc.store_scatter(dst, [dtv[g], dr_vec, jv], ws[g])
                return carry

            lax.fori_loop(0, (BW - JS) // L, per_blk, 0)

        def step(t, row, p, o):
            g_desc(row, p).wait()
            @pl.when(row < TH - 1)
            def _():
                g_desc(row + 1, p ^ 1).start()
            @pl.when(t >= 2)
            def _():
                o_desc(t - 2, o).wait()
            streamed_transpose(rows[p], outs[o], t)
            vldidx_transpose(row, outs[o], t)
            o_desc(t, o).start()

        def half(h):
            stage_half(h)
            g_desc(0, 0).start()

            def outer(u, carry):
                for i in range(2):
                    row = u * 2 + i
                    step(h * TH + row, row, i, i)
                return carry

            lax.fori_loop(0, TH // 2, outer, 0)

        half(0)
        half(1)

        for o in range(2):
            o_desc(T - 2 + o, o).wait()

    return k(xt, aug_table, pos_table)


def kernel(x, start_token, end_token, tok_table, pos_table):
    xt = jnp.swapaxes(x, 0, 1)
    aug_table = jnp.concatenate(
        [tok_table, jnp.float32(-5.0) - pos_table], axis=0)
    out5 = _sc_embed(xt, aug_table, pos_table)
    return jnp.transpose(out5, (2, 4, 0, 1, 3)).reshape(B, T, D)


# speedup vs baseline: 1.5914x; 1.0006x over previous
import functools

import jax
import jax.numpy as jnp
from jax import lax
from jax.experimental import pallas as pl
from jax.experimental.pallas import tpu as pltpu
from jax.experimental.pallas import tpu_sc as plsc

B, T, V, D = 4096, 200, 1000, 64
L = 16
NW = 32
BW = B // NW
JS = 80
TH = T // 2
VA = V + T


def _sc_embed(xt, aug_table, pos_table):
    mesh = plsc.VectorSubcoreMesh(core_axis_name="c", subcore_axis_name="s")

    @functools.partial(
        pl.kernel,
        mesh=mesh,
        compiler_params=pltpu.CompilerParams(
            use_tc_tiling_on_sc=False, needs_layout_passes=False),
        out_type=jax.ShapeDtypeStruct((T, D // 8, NW, 8, BW), jnp.float32),
        scratch_types=(
            [pltpu.VMEM((TH, BW), jnp.int32)]
            + [pltpu.VMEM((JS, D), jnp.float32) for _ in range(2)]
            + [pltpu.VMEM((D // 8, 8, BW + 1), jnp.float32) for _ in range(2)]
            + [pltpu.VMEM((T, D), jnp.float32)]
            + [pltpu.VMEM((VA, D), jnp.float32)]
            + [pltpu.SemaphoreType.DMA for _ in range(4)]
        ),
    )
    def k(x_hbm, aug_hbm, pos_hbm, out_hbm, idx_h, r0, r1, o0, o1,
          pos_v, aug_v, g0, g1, s0, s1):
        rows = (r0, r1)
        outs = (o0, o1)
        gsem = (g0, g1)
        osem = (s0, s1)
        wid = lax.axis_index("s") * 2 + lax.axis_index("c")
        b0 = wid * BW

        pltpu.sync_copy(aug_hbm, aug_v)
        pltpu.sync_copy(pos_hbm, pos_v)

        iota = lax.iota(jnp.int32, L)
        dr_vec = jnp.where(iota >= 8, iota - 8, iota)
        dt_half = jnp.where(iota >= 8, 1, 0)
        dtv = [dt_half + 2 * g for g in range(D // L)]
        dvec = [iota + g * L for g in range(D // L)]
        zv = jnp.zeros((L,), jnp.int32)

        def stage_half(h):
            pltpu.sync_copy(x_hbm.at[pl.ds(h * TH, TH), pl.ds(b0, BW)],
                            idx_h)

            def remap(r, carry):
                for kk in range(BW // L):
                    v = idx_h[r, pl.ds(kk * L, L)]
                    idx_h[r, pl.ds(kk * L, L)] = jnp.where(
                        v == 2, (V + h * TH) + r, v)
                return carry

            lax.fori_loop(0, TH, remap, 0, unroll=2)

        def g_desc(row, p):
            return pltpu.make_async_copy(
                aug_hbm.at[idx_h.at[row, pl.ds(0, JS)]], rows[p], gsem[p])

        def o_desc(t, o):
            return pltpu.make_async_copy(
                outs[o].at[:, :, pl.ds(0, BW)], out_hbm.at[t, :, wid],
                osem[o])

        def streamed_transpose(src, dst, t):
            pv = [pos_v[t, pl.ds(g * L, L)] for g in range(D // L)]

            def per_j(j, carry):
                jv = zv + j
                vs = [src[j, pl.ds(g * L, L)] for g in range(D // L)]
                ws = [vs[g] + pv[g] for g in range(D // L)]
                for g in range(D // L):
                    plsc.store_scatter(dst, [dtv[g], dr_vec, jv], ws[g])
                return carry

            lax.fori_loop(0, JS, per_j, 0, unroll=4)

        def vldidx_transpose(row, dst, t):
            pv = [pos_v[t, pl.ds(g * L, L)] for g in range(D // L)]

            def per_blk(q, carry):
                j0 = (JS // L) + q
                ivv = idx_h[row, pl.ds(j0 * L, L)]
                for l in range(L):
                    iv = jnp.full((L,), ivv[l], jnp.int32)
                    jv = zv + (j0 * L + l)
                    vs = [plsc.load_gather(aug_v, [iv, dvec[g]])
                          for g in range(D // L)]
                    ws = [vs[g] + pv[g] for g in range(D // L)]
                    for g in range(D // L):
                        plsc.store_scatter(dst, [dtv[g], dr_vec, jv], ws[g])
                return carry

            lax.fori_loop(0, (BW - JS) // L, per_blk, 0)

        def step(t, row, p, o):
            g_desc(row, p).wait()
            @pl.when(row < TH - 1)
            def _():
                g_desc(row + 1, p ^ 1).start()
            @pl.when(t >= 2)
            def _():
                o_desc(t - 2, o).wait()
            streamed_transpose(rows[p], outs[o], t)
            vldidx_transpose(row, outs[o], t)
            o_desc(t, o).start()

        def half(h):
            stage_half(h)
            g_desc(0, 0).start()

            def outer(u, carry):
                for i in range(2):
                    row = u * 2 + i
                    step(h * TH + row, row, i, i)
                return carry

            lax.fori_loop(0, TH // 2, outer, 0)

        half(0)
        half(1)

        for o in range(2):
            o_desc(T - 2 + o, o).wait()

    return k(xt, aug_table, pos_table)


def kernel(x, start_token, end_token, tok_table, pos_table):
    xt = jnp.swapaxes(x, 0, 1)
    aug_table = jnp.concatenate(
        [tok_table, jnp.float32(-5.0) - pos_table], axis=0)
    out5 = _sc_embed(xt, aug_table, pos_table)
    return jnp.transpose(out5, (2, 4, 0, 1, 3)).reshape(B, T, D)


# speedup vs baseline: 1.6579x; 1.0418x over previous
import functools

import jax
import jax.numpy as jnp
from jax import lax
from jax.experimental import pallas as pl
from jax.experimental.pallas import tpu as pltpu
from jax.experimental.pallas import tpu_sc as plsc

B, T, V, D = 4096, 200, 1000, 64
L = 16
NW = 32
BW = B // NW
NG = 4
NO = 2


def _sc_embed(xt, aug_table, pos_table):
    mesh = plsc.VectorSubcoreMesh(core_axis_name="c", subcore_axis_name="s")

    @functools.partial(
        pl.kernel,
        mesh=mesh,
        compiler_params=pltpu.CompilerParams(use_tc_tiling_on_sc=False, needs_layout_passes=False),
        out_type=jax.ShapeDtypeStruct((T, D // 8, NW, 8, BW), jnp.float32),
        scratch_types=(
            [pltpu.VMEM((T, BW), jnp.int32)]
            + [pltpu.VMEM((BW, D), jnp.float32) for _ in range(NG)]
            + [pltpu.VMEM((D // 8, 8, BW + 1), jnp.float32) for _ in range(NO)]
            + [pltpu.VMEM((T, D), jnp.float32)]
            + [pltpu.SemaphoreType.DMA for _ in range(NG + NO)]
        ),
    )
    def k(x_hbm, aug_hbm, pos_hbm, out_hbm, idx_v, r0, r1, r2, r3, o0, o1,
          pos_v, g0, g1, g2, g3, s0, s1):
        rows = (r0, r1, r2, r3)
        outs = (o0, o1)
        gsem = (g0, g1, g2, g3)
        osem = (s0, s1)
        wid = lax.axis_index("s") * 2 + lax.axis_index("c")
        b0 = wid * BW

        pltpu.sync_copy(pos_hbm, pos_v)
        pltpu.sync_copy(x_hbm.at[:, pl.ds(b0, BW)], idx_v)

        def remap(t, carry):
            for kk in range(BW // L):
                v = idx_v[t, pl.ds(kk * L, L)]
                idx_v[t, pl.ds(kk * L, L)] = jnp.where(v == 2, t + V, v)
            return carry

        lax.fori_loop(0, T, remap, 0, unroll=2)

        def g_desc(t, g):
            return pltpu.make_async_copy(
                aug_hbm.at[idx_v.at[t]], rows[g], gsem[g])

        def o_desc(t, o):
            return pltpu.make_async_copy(
                outs[o].at[:, :, pl.ds(0, BW)], out_hbm.at[t, :, wid],
                osem[o])

        for g in range(NG - 1):
            g_desc(g, g).start()

        iota = lax.iota(jnp.int32, L)
        dr_vec = jnp.where(iota >= 8, iota - 8, iota)
        dt_half = jnp.where(iota >= 8, 1, 0)

        def transpose_add(src, dst, t):
            pv = [pos_v[t, pl.ds(g * L, L)] for g in range(D // L)]
            dtv = [dt_half + 2 * g for g in range(D // L)]

            def per_j(j, jj):
                vs = [src[j, pl.ds(g * L, L)] for g in range(D // L)]
                ws = [vs[g] + pv[g] for g in range(D // L)]
                for g in range(D // L):
                    plsc.store_scatter(dst, [dtv[g], dr_vec, jj], ws[g])
                return jj + 1

            lax.fori_loop(0, BW, per_j, jnp.zeros((L,), jnp.int32), unroll=4)

        def step(t, g, o):
            g_desc(t, g).wait()
            @pl.when(t >= NO)
            def _():
                o_desc(t - NO, o).wait()
            transpose_add(rows[g], outs[o], t)
            o_desc(t, o).start()
            tn = t + NG - 1
            @pl.when(tn < T)
            def _():
                g_desc(tn, (g + NG - 1) % NG).start()

        def outer(u, carry):
            t0 = u * (NG * NO)
            for i in range(NG * NO):
                step(t0 + i, i % NG, i % NO)
            return carry

        lax.fori_loop(0, T // (NG * NO), outer, 0)
        for i in range(T - (T // (NG * NO)) * (NG * NO)):
            step((T // (NG * NO)) * (NG * NO) + i, i % NG, i % NO)

        for o in range(NO):
            o_desc(T - NO + o, (T - NO + o) % NO).wait()

    return k(xt, aug_table, pos_table)


def kernel(x, start_token, end_token, tok_table, pos_table):
    xt = jnp.swapaxes(x, 0, 1)
    aug_table = jnp.concatenate(
        [tok_table, jnp.float32(-5.0) - pos_table], axis=0)
    out5 = _sc_embed(xt, aug_table, pos_table)
    return jnp.transpose(out5, (2, 4, 0, 1, 3)).reshape(B, T, D)
